# baseline (speedup 1.0000x reference)
import functools

import jax
import jax.numpy as jnp
from jax import lax
from jax.experimental import pallas as pl
from jax.experimental.pallas import tpu as pltpu
from jax.experimental.pallas import tpu_sc as plsc

_ATOM_FDIM = 128
_BOND_FDIM = 144
_HIDDEN = 256
_DEPTH = 3
_NA = 10000
_NB = 160000
_MAXNB = 16
_NM = 100

_NC = 2
_NS = 16
_L = 16
_NW = _NC * _NS

_APAD = 10240
_APW = _APAD // _NW
_CA = 8
_ASTEPS = _APW // _CA
_BPW = _NB // _NW
_CB = 128
_BSTEPS = _BPW // _CB
_BTAIL = _BPW - _BSTEPS * _CB


def _sc_mesh():
    return plsc.VectorSubcoreMesh(
        core_axis_name="c", subcore_axis_name="s",
        num_cores=_NC, num_subcores=_NS)


def _gathersum(msg, a2b_flat):
    @functools.partial(
        pl.kernel,
        out_type=jax.ShapeDtypeStruct((_APAD, _HIDDEN), jnp.float32),
        mesh=_sc_mesh(),
        scratch_types=[
            pltpu.VMEM((_CA * _MAXNB,), jnp.int32),
            pltpu.VMEM((_CA * _MAXNB, _HIDDEN), jnp.float32),
            pltpu.VMEM((_CA, _HIDDEN), jnp.float32),
            pltpu.SemaphoreType.DMA,
        ],
    )
    def k(msg_hbm, a2b_hbm, out_hbm, idx_v, rows_v, acc_v, sem):
        wid = lax.axis_index("s") * _NC + lax.axis_index("c")
        a_base = wid * _APW

        def step(i, carry):
            a0 = a_base + i * _CA
            pltpu.sync_copy(a2b_hbm.at[pl.ds(a0 * _MAXNB, _CA * _MAXNB)], idx_v)
            pltpu.async_copy(msg_hbm.at[idx_v], rows_v, sem).wait()

            def atom_body(a, c2):
                base = a * _MAXNB
                for c in range(_HIDDEN // _L):
                    s = pl.ds(c * _L, _L)
                    acc = rows_v[base, s]
                    for j in range(1, _MAXNB):
                        acc = acc + rows_v[base + j, s]
                    acc_v[a, s] = acc
                return c2

            lax.fori_loop(0, _CA, atom_body, 0)
            pltpu.sync_copy(acc_v, out_hbm.at[pl.ds(a0, _CA)])
            return carry

        lax.fori_loop(0, _ASTEPS, step, 0)

    return k(msg, a2b_flat)


def _combine(inp, ah, mh, b2a, b2revb):
    @functools.partial(
        pl.kernel,
        out_type=jax.ShapeDtypeStruct((_NB, _HIDDEN), jnp.float32),
        mesh=_sc_mesh(),
        scratch_types=[
            pltpu.VMEM((_CB,), jnp.int32),
            pltpu.VMEM((_CB,), jnp.int32),
            pltpu.VMEM((_CB, _HIDDEN), jnp.float32),
            pltpu.VMEM((_CB, _HIDDEN), jnp.float32),
            pltpu.VMEM((_CB, _HIDDEN), jnp.float32),
            pltpu.SemaphoreType.DMA,
        ],
    )
    def k(inp_hbm, ah_hbm, mh_hbm, b2a_hbm, b2revb_hbm, out_hbm,
          ia_v, ir_v, ah_v, mh_v, x_v, sem):
        wid = lax.axis_index("s") * _NC + lax.axis_index("c")
        b_base = wid * _BPW

        def chunk(b0, n):
            pltpu.sync_copy(b2a_hbm.at[pl.ds(b0, n)], ia_v.at[pl.ds(0, n)])
            pltpu.sync_copy(b2revb_hbm.at[pl.ds(b0, n)], ir_v.at[pl.ds(0, n)])
            pltpu.async_copy(ah_hbm.at[ia_v], ah_v, sem).wait()
            pltpu.async_copy(mh_hbm.at[ir_v], mh_v, sem).wait()
            pltpu.sync_copy(inp_hbm.at[pl.ds(b0, n)], x_v.at[pl.ds(0, n)])

            def row_body(r, c2):
                for c in range(_HIDDEN // _L):
                    s = pl.ds(c * _L, _L)
                    x_v[r, s] = jnp.maximum(
                        x_v[r, s] + ah_v[r, s] - mh_v[r, s], 0.0)
                return c2

            lax.fori_loop(0, n, row_body, 0)
            pltpu.sync_copy(x_v.at[pl.ds(0, n)], out_hbm.at[pl.ds(b0, n)])

        def step(i, carry):
            chunk(b_base + i * _CB, _CB)
            return carry

        lax.fori_loop(0, _BSTEPS, step, 0)
        if _BTAIL:
            chunk(b_base + _BSTEPS * _CB, _BTAIL)

    return k(inp, ah, mh, b2a, b2revb)


def _mm(x, wt, rb, with_relu_copy):
    n, kdim = x.shape
    h = wt.shape[1]
    grid = (n // rb,)
    in_specs = [
        pl.BlockSpec((rb, kdim), lambda i: (i, 0)),
        pl.BlockSpec((kdim, h), lambda i: (0, 0)),
    ]
    if with_relu_copy:
        def body(x_ref, w_ref, y_ref, r_ref):
            y = jnp.dot(x_ref[...], w_ref[...],
                        preferred_element_type=jnp.float32)
            y_ref[...] = y
            r_ref[...] = jnp.maximum(y, 0.0)

        return pl.pallas_call(
            body, grid=grid, in_specs=in_specs,
            out_specs=[pl.BlockSpec((rb, h), lambda i: (i, 0))] * 2,
            out_shape=[jax.ShapeDtypeStruct((n, h), jnp.float32)] * 2,
        )(x, wt)

    def body(x_ref, w_ref, y_ref):
        y_ref[...] = jnp.dot(x_ref[...], w_ref[...],
                             preferred_element_type=jnp.float32)

    return pl.pallas_call(
        body, grid=grid, in_specs=in_specs,
        out_specs=pl.BlockSpec((rb, h), lambda i: (i, 0)),
        out_shape=jax.ShapeDtypeStruct((n, h), jnp.float32),
    )(x, wt)


_RBF = 2000


def _finalize(f_atoms, a_msg, seg3, wo1t, wo2t, bo2):
    grid = (_NA // _RBF,)

    def body(fa_ref, am_ref, seg_ref, w1_ref, w2_ref, bo_ref, out_ref,
             acc, cnt):
        i = pl.program_id(0)

        @pl.when(i == 0)
        def _():
            acc[...] = jnp.zeros_like(acc)
            cnt[...] = jnp.zeros_like(cnt)

        h = jnp.dot(fa_ref[...], w1_ref[...],
                    preferred_element_type=jnp.float32)
        h += jnp.dot(am_ref[...], w2_ref[...],
                     preferred_element_type=jnp.float32)
        h = jnp.maximum(h + bo_ref[...], 0.0)
        seg = seg_ref[0, 0, :]
        onehot = (seg[None, :] == lax.broadcasted_iota(
            jnp.int32, (_NM, _RBF), 0)).astype(jnp.float32)
        acc[...] += jnp.dot(onehot, h, preferred_element_type=jnp.float32)
        cnt[...] += jnp.sum(onehot, axis=1, keepdims=True)

        @pl.when(i == pl.num_programs(0) - 1)
        def _():
            c = cnt[...]
            out_ref[...] = jnp.where(
                c > 0.0, acc[...] / jnp.maximum(c, 1.0), 0.0)

    return pl.pallas_call(
        body, grid=grid,
        in_specs=[
            pl.BlockSpec((_RBF, _ATOM_FDIM), lambda i: (i, 0)),
            pl.BlockSpec((_RBF, _HIDDEN), lambda i: (i, 0)),
            pl.BlockSpec((1, 1, _RBF), lambda i: (i, 0, 0)),
            pl.BlockSpec((_ATOM_FDIM, _HIDDEN), lambda i: (0, 0)),
            pl.BlockSpec((_HIDDEN, _HIDDEN), lambda i: (0, 0)),
            pl.BlockSpec((1, _HIDDEN), lambda i: (0, 0)),
        ],
        out_specs=pl.BlockSpec((_NM, _HIDDEN), lambda i: (0, 0)),
        out_shape=jax.ShapeDtypeStruct((_NM, _HIDDEN), jnp.float32),
        scratch_shapes=[
            pltpu.VMEM((_NM, _HIDDEN), jnp.float32),
            pltpu.VMEM((_NM, 1), jnp.float32),
        ],
    )(f_atoms, a_msg, seg3, wo1t, wo2t, bo2)


def kernel(f_atoms, f_bonds, a2b, b2a, b2revb, segment_ids, W_i, W_h, W_o, b_o):
    a2b_flat = jnp.pad(a2b.astype(jnp.int32).reshape(-1),
                       (0, (_APAD - _NA) * _MAXNB))
    b2a = b2a.astype(jnp.int32)
    b2revb = b2revb.astype(jnp.int32)
    seg3 = segment_ids.astype(jnp.int32).reshape(_NA // _RBF, 1, _RBF)
    wit = W_i.T
    wht = W_h.T
    wo1t = W_o[:, :_ATOM_FDIM].T
    wo2t = W_o[:, _ATOM_FDIM:].T
    bo2 = b_o.reshape(1, _HIDDEN)

    inp, msg = _mm(f_bonds, wit, 3200, True)
    for _ in range(_DEPTH - 1):
        am = _gathersum(msg, a2b_flat)
        mh = _mm(msg, wht, 3200, False)
        ah = _mm(am, wht, 1024, False)
        msg = _combine(inp, ah, mh, b2a, b2revb)
    am = _gathersum(msg, a2b_flat)[:_NA]
    return _finalize(f_atoms, am, seg3, wo1t, wo2t, bo2)


# speedup vs baseline: 1.4209x; 1.4209x over previous
import functools

import jax
import jax.numpy as jnp
from jax import lax
from jax.experimental import pallas as pl
from jax.experimental.pallas import tpu as pltpu
from jax.experimental.pallas import tpu_sc as plsc

_ATOM_FDIM = 128
_BOND_FDIM = 144
_HIDDEN = 256
_DEPTH = 3
_NA = 10000
_NB = 160000
_MAXNB = 16
_NM = 100

_NC = 2
_NS = 16
_L = 16
_NW = _NC * _NS

_APAD = 10240
_APW = _APAD // _NW
_CA = 4
_AROWS = _CA * _MAXNB
_ASTEPS = _APW // _CA
_BPW = _NB // _NW
_CB = 64
_BSTEPS = _BPW // _CB
_BTAIL = _BPW - _BSTEPS * _CB


def _sc_mesh():
    return plsc.VectorSubcoreMesh(
        core_axis_name="c", subcore_axis_name="s",
        num_cores=_NC, num_subcores=_NS)


def _wid():
    return lax.axis_index("s") * _NC + lax.axis_index("c")


def _gathersum(msg, a2b_flat):
    @functools.partial(
        pl.kernel,
        out_type=jax.ShapeDtypeStruct((_APAD, _HIDDEN), jnp.float32),
        mesh=_sc_mesh(),
        scratch_types=[
            pltpu.VMEM((_APW * _MAXNB,), jnp.int32),
            pltpu.VMEM((_AROWS, _HIDDEN), jnp.float32),
            pltpu.VMEM((_AROWS, _HIDDEN), jnp.float32),
            pltpu.VMEM((_APW, _HIDDEN), jnp.float32),
            pltpu.SemaphoreType.DMA,
            pltpu.SemaphoreType.DMA,
            pltpu.SemaphoreType.DMA,
        ],
    )
    def k(msg_hbm, a2b_hbm, out_hbm, idx_v, rows0, rows1, out_v,
          gsem0, gsem1, isem):
        rows = (rows0, rows1)
        gsem = (gsem0, gsem1)
        a_base = _wid() * _APW

        pltpu.sync_copy(a2b_hbm.at[pl.ds(a_base * _MAXNB, _APW * _MAXNB)],
                        idx_v)

        def issue(s, b):
            src = msg_hbm.at[idx_v.at[pl.ds(s * _AROWS, _AROWS)]]
            pltpu.async_copy(src, rows[b], gsem[b])

        def wait_rows(b):
            pltpu.make_async_copy(
                msg_hbm.at[idx_v.at[pl.ds(0, _AROWS)]], rows[b],
                gsem[b]).wait()

        def compute(s, b):
            def atom_body(a, c2):
                base = a * _MAXNB
                for c in range(_HIDDEN // _L):
                    sl = pl.ds(c * _L, _L)
                    acc = rows[b][base, sl]
                    for j in range(1, _MAXNB):
                        acc = acc + rows[b][base + j, sl]
                    out_v[s * _CA + a, sl] = acc
                return c2
            lax.fori_loop(0, _CA, atom_body, 0)

        issue(0, 0)

        def pair(s2, carry):
            s0 = 2 * s2
            issue(s0 + 1, 1)
            wait_rows(0)
            compute(s0, 0)
            issue(s0 + 2, 0)
            wait_rows(1)
            compute(s0 + 1, 1)
            return carry

        lax.fori_loop(0, (_ASTEPS - 2) // 2, pair, 0)
        s0 = _ASTEPS - 2
        issue(s0 + 1, 1)
        wait_rows(0)
        compute(s0, 0)
        wait_rows(1)
        compute(s0 + 1, 1)

        pltpu.async_copy(out_v, out_hbm.at[pl.ds(a_base, _APW)], isem)
        pltpu.make_async_copy(
            out_v, out_hbm.at[pl.ds(a_base, _APW)], isem).wait()

    return k(msg, a2b_flat)


def _diff(am, msg, b2a, b2revb):
    @functools.partial(
        pl.kernel,
        out_type=jax.ShapeDtypeStruct((_NB, _HIDDEN), jnp.float32),
        mesh=_sc_mesh(),
        scratch_types=[
            pltpu.VMEM((_BPW,), jnp.int32),
            pltpu.VMEM((_BPW,), jnp.int32),
            pltpu.VMEM((_CB, _HIDDEN), jnp.float32),
            pltpu.VMEM((_CB, _HIDDEN), jnp.float32),
            pltpu.VMEM((_CB, _HIDDEN), jnp.float32),
            pltpu.VMEM((_CB, _HIDDEN), jnp.float32),
            pltpu.SemaphoreType.DMA,
            pltpu.SemaphoreType.DMA,
            pltpu.SemaphoreType.DMA,
            pltpu.SemaphoreType.DMA,
            pltpu.SemaphoreType.DMA,
            pltpu.SemaphoreType.DMA,
        ],
    )
    def k(am_hbm, msg_hbm, b2a_hbm, b2revb_hbm, out_hbm,
          ia_v, ir_v, ag0, ag1, mg0, mg1,
          asem0, asem1, msem0, msem1, osem0, osem1):
        ag = (ag0, ag1)
        mg = (mg0, mg1)
        asem = (asem0, asem1)
        msem = (msem0, msem1)
        osem = (osem0, osem1)
        b_base = _wid() * _BPW

        pltpu.sync_copy(b2a_hbm.at[pl.ds(b_base, _BPW)], ia_v)
        pltpu.sync_copy(b2revb_hbm.at[pl.ds(b_base, _BPW)], ir_v)

        def issue(s, b, n):
            pltpu.async_copy(
                am_hbm.at[ia_v.at[pl.ds(s * _CB, n)]],
                ag[b].at[pl.ds(0, n)], asem[b])
            pltpu.async_copy(
                msg_hbm.at[ir_v.at[pl.ds(s * _CB, n)]],
                mg[b].at[pl.ds(0, n)], msem[b])

        def wait_rows(b, n):
            pltpu.make_async_copy(
                am_hbm.at[ia_v.at[pl.ds(0, n)]],
                ag[b].at[pl.ds(0, n)], asem[b]).wait()
            pltpu.make_async_copy(
                msg_hbm.at[ir_v.at[pl.ds(0, n)]],
                mg[b].at[pl.ds(0, n)], msem[b]).wait()

        def compute(b, n):
            def row_body(r, c2):
                for c in range(_HIDDEN // _L):
                    sl = pl.ds(c * _L, _L)
                    ag[b][r, sl] = ag[b][r, sl] - mg[b][r, sl]
                return c2
            lax.fori_loop(0, n, row_body, 0)

        def writeback(s, b, n):
            pltpu.async_copy(
                ag[b].at[pl.ds(0, n)],
                out_hbm.at[pl.ds(b_base + s * _CB, n)], osem[b])

        def wait_wb(b, n):
            pltpu.make_async_copy(
                ag[b].at[pl.ds(0, n)],
                out_hbm.at[pl.ds(b_base, n)], osem[b]).wait()

        issue(0, 0, _CB)

        def pair(s2, carry):
            s0 = 2 * s2

            @pl.when(s2 > 0)
            def _():
                wait_wb(1, _CB)
            issue(s0 + 1, 1, _CB)
            wait_rows(0, _CB)
            compute(0, _CB)
            writeback(s0, 0, _CB)

            wait_wb(0, _CB)
            issue(s0 + 2, 0, _CB)
            wait_rows(1, _CB)
            compute(1, _CB)
            writeback(s0 + 1, 1, _CB)
            return carry

        lax.fori_loop(0, (_BSTEPS - 2) // 2, pair, 0)

        s0 = _BSTEPS - 2
        wait_wb(1, _CB)
        issue(s0 + 1, 1, _CB)
        wait_rows(0, _CB)
        compute(0, _CB)
        writeback(s0, 0, _CB)

        wait_wb(0, _CB)
        issue(_BSTEPS, 0, _BTAIL)
        wait_rows(1, _CB)
        compute(1, _CB)
        writeback(s0 + 1, 1, _CB)

        wait_rows(0, _BTAIL)
        compute(0, _BTAIL)
        pltpu.sync_copy(
            ag[0].at[pl.ds(0, _BTAIL)],
            out_hbm.at[pl.ds(b_base + _BSTEPS * _CB, _BTAIL)])

        wait_wb(1, _CB)

    return k(am, msg, b2a, b2revb)


def _mm(x, wt, rb, with_relu_copy):
    n, kdim = x.shape
    h = wt.shape[1]
    grid = (n // rb,)
    in_specs = [
        pl.BlockSpec((rb, kdim), lambda i: (i, 0)),
        pl.BlockSpec((kdim, h), lambda i: (0, 0)),
    ]
    if with_relu_copy:
        def body(x_ref, w_ref, y_ref, r_ref):
            y = jnp.dot(x_ref[...], w_ref[...],
                        preferred_element_type=jnp.float32)
            y_ref[...] = y
            r_ref[...] = jnp.maximum(y, 0.0)

        return pl.pallas_call(
            body, grid=grid, in_specs=in_specs,
            out_specs=[pl.BlockSpec((rb, h), lambda i: (i, 0))] * 2,
            out_shape=[jax.ShapeDtypeStruct((n, h), jnp.float32)] * 2,
        )(x, wt)

    def body(x_ref, w_ref, y_ref):
        y_ref[...] = jnp.dot(x_ref[...], w_ref[...],
                             preferred_element_type=jnp.float32)

    return pl.pallas_call(
        body, grid=grid, in_specs=in_specs,
        out_specs=pl.BlockSpec((rb, h), lambda i: (i, 0)),
        out_shape=jax.ShapeDtypeStruct((n, h), jnp.float32),
    )(x, wt)


def _mm_add_relu(t, wt, inp, rb):
    n = t.shape[0]
    h = wt.shape[1]

    def body(t_ref, w_ref, i_ref, o_ref):
        y = jnp.dot(t_ref[...], w_ref[...], preferred_element_type=jnp.float32)
        o_ref[...] = jnp.maximum(i_ref[...] + y, 0.0)

    return pl.pallas_call(
        body, grid=(n // rb,),
        in_specs=[
            pl.BlockSpec((rb, _HIDDEN), lambda i: (i, 0)),
            pl.BlockSpec((_HIDDEN, h), lambda i: (0, 0)),
            pl.BlockSpec((rb, h), lambda i: (i, 0)),
        ],
        out_specs=pl.BlockSpec((rb, h), lambda i: (i, 0)),
        out_shape=jax.ShapeDtypeStruct((n, h), jnp.float32),
    )(t, wt, inp)


_RBF = 2000


def _finalize(f_atoms, a_msg, seg3, wo1t, wo2t, bo2):
    grid = (_NA // _RBF,)

    def body(fa_ref, am_ref, seg_ref, w1_ref, w2_ref, bo_ref, out_ref,
             acc, cnt):
        i = pl.program_id(0)

        @pl.when(i == 0)
        def _():
            acc[...] = jnp.zeros_like(acc)
            cnt[...] = jnp.zeros_like(cnt)

        h = jnp.dot(fa_ref[...], w1_ref[...],
                    preferred_element_type=jnp.float32)
        h += jnp.dot(am_ref[...], w2_ref[...],
                     preferred_element_type=jnp.float32)
        h = jnp.maximum(h + bo_ref[...], 0.0)
        seg = seg_ref[0, 0, :]
        onehot = (seg[None, :] == lax.broadcasted_iota(
            jnp.int32, (_NM, _RBF), 0)).astype(jnp.float32)
        acc[...] += jnp.dot(onehot, h, preferred_element_type=jnp.float32)
        cnt[...] += jnp.sum(onehot, axis=1, keepdims=True)

        @pl.when(i == pl.num_programs(0) - 1)
        def _():
            c = cnt[...]
            out_ref[...] = jnp.where(
                c > 0.0, acc[...] / jnp.maximum(c, 1.0), 0.0)

    return pl.pallas_call(
        body, grid=grid,
        in_specs=[
            pl.BlockSpec((_RBF, _ATOM_FDIM), lambda i: (i, 0)),
            pl.BlockSpec((_RBF, _HIDDEN), lambda i: (i, 0)),
            pl.BlockSpec((1, 1, _RBF), lambda i: (i, 0, 0)),
            pl.BlockSpec((_ATOM_FDIM, _HIDDEN), lambda i: (0, 0)),
            pl.BlockSpec((_HIDDEN, _HIDDEN), lambda i: (0, 0)),
            pl.BlockSpec((1, _HIDDEN), lambda i: (0, 0)),
        ],
        out_specs=pl.BlockSpec((_NM, _HIDDEN), lambda i: (0, 0)),
        out_shape=jax.ShapeDtypeStruct((_NM, _HIDDEN), jnp.float32),
        scratch_shapes=[
            pltpu.VMEM((_NM, _HIDDEN), jnp.float32),
            pltpu.VMEM((_NM, 1), jnp.float32),
        ],
    )(f_atoms, a_msg, seg3, wo1t, wo2t, bo2)


def kernel(f_atoms, f_bonds, a2b, b2a, b2revb, segment_ids, W_i, W_h, W_o, b_o):
    a2b_flat = jnp.pad(a2b.astype(jnp.int32).reshape(-1),
                       (0, (_APAD - _NA) * _MAXNB))
    b2a = b2a.astype(jnp.int32)
    b2revb = b2revb.astype(jnp.int32)
    seg3 = segment_ids.astype(jnp.int32).reshape(_NA // _RBF, 1, _RBF)
    wit = W_i.T
    wht = W_h.T
    wo1t = W_o[:, :_ATOM_FDIM].T
    wo2t = W_o[:, _ATOM_FDIM:].T
    bo2 = b_o.reshape(1, _HIDDEN)

    inp, msg = _mm(f_bonds, wit, 3200, True)
    for _ in range(_DEPTH - 1):
        am = _gathersum(msg, a2b_flat)
        t = _diff(am, msg, b2a, b2revb)
        msg = _mm_add_relu(t, wht, inp, 3200)
    am = _gathersum(msg, a2b_flat)[:_NA]
    return _finalize(f_atoms, am, seg3, wo1t, wo2t, bo2)
